# baseline (speedup 1.0000x reference)
import functools

import jax
import jax.numpy as jnp
from jax import lax
from jax.experimental import pallas as pl
from jax.experimental.pallas import tpu as pltpu
from jax.experimental.pallas import tpu_sc as plsc

B = 32768
H = 768
H2 = 384
E = 64
TB = 4096

NC = 2
NS = 16
NW = NC * NS
RPW = B // NW
CR = 256
NG = CR // 16


def _logits_kernel(x_ref, w1_ref, b1_ref, w2_ref, b2_ref, out_ref):
    x = x_ref[...]
    h = jnp.dot(x, w1_ref[...], preferred_element_type=jnp.float32)
    h = jnp.maximum(h + b1_ref[...], 0.0)
    logits = jnp.dot(h, w2_ref[...], preferred_element_type=jnp.float32)
    out_ref[...] = logits + b2_ref[...]


def _tc_logits(query_repr, W1, b1r, W2, b2r):
    grid = (B // TB,)
    return pl.pallas_call(
        _logits_kernel,
        grid=grid,
        in_specs=[
            pl.BlockSpec((TB, H), lambda i: (i, 0)),
            pl.BlockSpec((H, H2), lambda i: (0, 0)),
            pl.BlockSpec((1, H2), lambda i: (0, 0)),
            pl.BlockSpec((H2, E), lambda i: (0, 0)),
            pl.BlockSpec((1, E), lambda i: (0, 0)),
        ],
        out_specs=pl.BlockSpec((TB, E), lambda i: (i, 0)),
        out_shape=jax.ShapeDtypeStruct((B, E), jnp.float32),
        compiler_params=pltpu.CompilerParams(
            dimension_semantics=("parallel",),
        ),
    )(query_repr, W1, b1r, W2, b2r)


def _sc_tail_body(logits_hbm, out_hbm, in_v, out_v):
    wid = lax.axis_index("s") * NC + lax.axis_index("c")
    base = wid * (RPW * E)
    lanes = lax.iota(jnp.int32, 16)
    row0 = lanes * E
    zero16 = jnp.zeros((16,), jnp.float32)
    for chunk in range(RPW // CR):
        cbase = base + chunk * (CR * E)
        pltpu.sync_copy(logits_hbm.at[pl.ds(cbase, CR * E)], in_v)

        def group_body(g, carry):
            gb = g * (16 * E)
            ridx = row0 + gb
            m1 = jnp.full((16,), -jnp.inf, jnp.float32)
            m2 = jnp.full((16,), -jnp.inf, jnp.float32)
            i1 = jnp.zeros((16,), jnp.int32)
            i2 = jnp.zeros((16,), jnp.int32)
            for e in range(E):
                ve = plsc.load_gather(in_v, [ridx + e])
                out_v[pl.ds(gb + e * 16, 16)] = zero16
                gt1 = ve > m1
                gt2 = jnp.logical_and(jnp.logical_not(gt1), ve > m2)
                ei = jnp.full((16,), e, jnp.int32)
                m2 = jnp.where(gt1, m1, jnp.where(gt2, ve, m2))
                i2 = jnp.where(gt1, i1, jnp.where(gt2, ei, i2))
                m1 = jnp.where(gt1, ve, m1)
                i1 = jnp.where(gt1, ei, i1)
            e2 = jnp.exp(m2 - m1)
            g1 = 1.0 / (1.0 + e2)
            g2 = e2 * g1
            plsc.store_scatter(out_v, [ridx + i1], g1)
            plsc.store_scatter(out_v, [ridx + i2], g2)
            return carry

        lax.fori_loop(0, NG, group_body, 0)
        pltpu.sync_copy(out_v, out_hbm.at[pl.ds(cbase, CR * E)])


_sc_tail = functools.partial(
    pl.kernel,
    mesh=plsc.VectorSubcoreMesh(core_axis_name="c", subcore_axis_name="s"),
    out_type=jax.ShapeDtypeStruct((B * E,), jnp.float32),
    scratch_types=[
        pltpu.VMEM((CR * E,), jnp.float32),
        pltpu.VMEM((CR * E,), jnp.float32),
    ],
    compiler_params=pltpu.CompilerParams(needs_layout_passes=False),
)(_sc_tail_body)


def kernel(query_repr, W1, b1, W2, b2):
    b1r = b1.reshape(1, H2)
    b2r = b2.reshape(1, E)
    logits = _tc_logits(query_repr, W1, b1r, W2, b2r)
    gates = _sc_tail(logits.reshape(B * E))
    return gates.reshape(B, E)
